# baseline (speedup 1.0000x reference)
import functools

import jax
import jax.numpy as jnp
from jax import lax
from jax.experimental import pallas as pl
from jax.experimental.pallas import tpu as pltpu
from jax.experimental.pallas import tpu_sc as plsc

B = 16384
D = 64
VW = 2 * D
NC = 2
NS = 16
NW = NC * NS
BPW = B // NW
GCHUNK = 128
NG = BPW // GCHUNK
LANES = 16

_mesh = plsc.VectorSubcoreMesh(core_axis_name="c", subcore_axis_name="s")


@functools.partial(
    pl.kernel,
    mesh=_mesh,
    out_type=jax.ShapeDtypeStruct((B // 2, VW), jnp.float32),
    scratch_types=[
        pltpu.VMEM((BPW,), jnp.int32),
        pltpu.VMEM((BPW,), jnp.int32),
        pltpu.VMEM((BPW, VW), jnp.float32),
        pltpu.VMEM((BPW // 2, VW), jnp.float32),
        pltpu.SemaphoreType.DMA,
    ],
    compiler_params=pltpu.CompilerParams(needs_layout_passes=False),
)
def _gather_tanh(idx_hbm, table_hbm, out_hbm, idx_v, idx2_v, rows_v, out_v, sem):
    wid = lax.axis_index("s") * NC + lax.axis_index("c")
    base = wid * BPW
    vbase = wid * (BPW // 2)

    pltpu.sync_copy(idx_hbm.at[pl.ds(base, BPW)], idx_v)

    for s in range(BPW // LANES):
        sl = pl.ds(s * LANES, LANES)
        idx2_v[sl] = idx_v[sl] >> 1

    copies = []
    for g in range(NG):
        copies.append(
            pltpu.async_copy(
                table_hbm.at[idx2_v.at[pl.ds(g * GCHUNK, GCHUNK)]],
                rows_v.at[pl.ds(g * GCHUNK, GCHUNK), :],
                sem,
            )
        )

    lane = lax.iota(jnp.int32, LANES)

    for g in range(NG):
        copies[g].wait()

        def blk(b, _, g=g):
            kbase = g * GCHUNK + b * LANES
            kvec = kbase + lane
            idxs = idx_v[pl.ds(kbase, LANES)]
            p64 = (idxs & 1) * D
            jvec = kvec >> 1
            q64 = (kvec & 1) * D

            def cloop(c, _):
                v = plsc.load_gather(rows_v, [kvec, p64 + c])
                e = jnp.exp(v * 2.0)
                y = 1.0 - 2.0 / (e + 1.0)
                plsc.store_scatter(out_v, [jvec, q64 + c], y)
                return 0

            lax.fori_loop(0, D, cloop, 0)
            return 0

        lax.fori_loop(0, GCHUNK // LANES, blk, 0)

    pltpu.sync_copy(out_v, out_hbm.at[pl.ds(vbase, BPW // 2), :])


def kernel(topics, table):
    out = _gather_tanh(topics.astype(jnp.int32), table.reshape(-1, VW))
    return out.reshape(B, 1, D)


# speedup vs baseline: 1.5397x; 1.5397x over previous
import functools

import jax
import jax.numpy as jnp
from jax import lax
from jax.experimental import pallas as pl
from jax.experimental.pallas import tpu as pltpu
from jax.experimental.pallas import tpu_sc as plsc

B = 16384
D = 64
NC = 2
NS = 16
KPT = B // NS
JPC = D // NC
ROWS = 1000000
CUT0 = 512000
CUT1 = 999936
LEN0 = CUT0
LEN1 = CUT1 - CUT0
SLC0 = LEN0 // NS
SLC1 = 30464
SLC1_LAST = LEN1 - 15 * SLC1
NTAIL = ROWS - CUT1
LANES = 16
GCH = 128
NG = KPT // GCH

_mesh = plsc.VectorSubcoreMesh(core_axis_name="c", subcore_axis_name="s")


@functools.partial(
    pl.kernel,
    mesh=_mesh,
    out_type=jax.ShapeDtypeStruct((D, B), jnp.float32),
    scratch_types=[
        pltpu.VMEM((KPT,), jnp.int32),
        pltpu.VMEM((KPT,), jnp.int32),
        pltpu.VMEM((KPT,), jnp.float32),
        pltpu.VMEM((JPC, KPT), jnp.float32),
        pltpu.VMEM((NTAIL * D,), jnp.float32),
        pltpu.VMEM_SHARED((LEN0,), jnp.float32),
        pltpu.VMEM_SHARED((LEN1,), jnp.float32),
        pltpu.SemaphoreType.DMA,
        pltpu.SemaphoreType.DMA,
        pltpu.SemaphoreType.DMA,
    ],
    compiler_params=pltpu.CompilerParams(needs_layout_passes=False),
)
def _gather_tanh(idx_hbm, table_hbm, tail_hbm, out_hbm, idx_v, rel_v,
                 tmp_v, acc_v, tail_v, ring0_sh, ring1_sh, sem0, sem1,
                 gsem):
    cid = lax.axis_index("c")
    sid = lax.axis_index("s")

    pltpu.sync_copy(idx_hbm.at[pl.ds(sid * KPT, KPT)], idx_v)
    pltpu.sync_copy(tail_hbm, tail_v)

    def fill(m, u, sem):
        o = 4 * cid + (m >> 3)
        b = m & 7
        src = table_hbm.at[o, b]
        if u == 0:
            pltpu.async_copy(
                src.at[pl.ds(sid * SLC0, SLC0)],
                ring0_sh.at[pl.ds(sid * SLC0, SLC0)],
                sem,
            )
        else:
            @pl.when(sid < NS - 1)
            def _():
                pltpu.async_copy(
                    src.at[pl.ds(CUT0 + sid * SLC1, SLC1)],
                    ring1_sh.at[pl.ds(sid * SLC1, SLC1)],
                    sem,
                )

            @pl.when(sid == NS - 1)
            def _():
                pltpu.async_copy(
                    src.at[pl.ds(CUT0 + 15 * SLC1, SLC1_LAST)],
                    ring1_sh.at[pl.ds(15 * SLC1, SLC1_LAST)],
                    sem,
                )

    def wait_fill(u, sem):
        if u == 0:
            pltpu.make_async_copy(
                table_hbm.at[0, 0, pl.ds(0, SLC0)],
                ring0_sh.at[pl.ds(0, SLC0)],
                sem,
            ).wait()
        else:
            @pl.when(sid < NS - 1)
            def _():
                pltpu.make_async_copy(
                    table_hbm.at[0, 0, pl.ds(0, SLC1)],
                    ring1_sh.at[pl.ds(0, SLC1)],
                    sem,
                ).wait()

            @pl.when(sid == NS - 1)
            def _():
                pltpu.make_async_copy(
                    table_hbm.at[0, 0, pl.ds(0, SLC1_LAST)],
                    ring1_sh.at[pl.ds(0, SLC1_LAST)],
                    sem,
                ).wait()

    def process(m, u):
        ring = ring0_sh if u == 0 else ring1_sh
        lo = 0 if u == 0 else CUT0
        ln = LEN0 if u == 0 else LEN1

        for s in range(KPT // LANES):
            sl = pl.ds(s * LANES, LANES)
            rel = idx_v[sl] - lo
            rel_v[sl] = jnp.minimum(jnp.maximum(rel, 0), ln - 1)

        for g in range(NG):
            pltpu.async_copy(
                ring.at[rel_v.at[pl.ds(g * GCH, GCH)]],
                tmp_v.at[pl.ds(g * GCH, GCH)],
                gsem,
            )
        pltpu.make_async_copy(
            table_hbm.at[0, 0, pl.ds(0, KPT)], tmp_v, gsem
        ).wait()

        jrow = D // NC * cid + m
        for s in range(KPT // LANES):
            sl = pl.ds(s * LANES, LANES)
            x = tmp_v[sl]
            e = jnp.exp(x * 2.0)
            y = 1.0 - 2.0 / (e + 1.0)
            if u == 0:
                acc_v[m, sl] = y
            else:
                t = idx_v[sl]
                toff = jrow * NTAIL + (t - CUT1)
                toff = jnp.minimum(jnp.maximum(toff, 0), NTAIL * D - 1)
                tv = plsc.load_gather(tail_v, [toff])
                et = jnp.exp(tv * 2.0)
                yt = 1.0 - 2.0 / (et + 1.0)
                y = jnp.where(t >= CUT1, yt, y)
                acc_v[m, sl] = jnp.where(t >= CUT0, y, acc_v[m, sl])

    fill(0, 0, sem0)
    fill(0, 1, sem1)

    def stage_pair(m, _):
        wait_fill(0, sem0)
        plsc.subcore_barrier()
        process(m, 0)
        plsc.subcore_barrier()

        @pl.when(m < JPC - 1)
        def _():
            fill(m + 1, 0, sem0)

        wait_fill(1, sem1)
        plsc.subcore_barrier()
        process(m, 1)
        plsc.subcore_barrier()

        @pl.when(m < JPC - 1)
        def _():
            fill(m + 1, 1, sem1)

        return 0

    lax.fori_loop(0, JPC, stage_pair, 0)

    pltpu.sync_copy(
        acc_v,
        out_hbm.at[pl.ds(JPC * cid, JPC), pl.ds(sid * KPT, KPT)],
    )


def kernel(topics, table):
    tail = table[CUT1:].T.reshape(-1)
    out_t = _gather_tanh(
        topics.astype(jnp.int32), table.T.reshape(8, 8, ROWS), tail
    )
    return out_t.T.reshape(B, 1, D)
